# Initial kernel scaffold; baseline (speedup 1.0000x reference)
import jax
import jax.numpy as jnp
from jax.experimental import pallas as pl


def kernel(target_idx, context_idx, negative_idx, target_embeddings, context_embeddings):
    raise NotImplementedError("write your pallas kernel here")


# speedup vs baseline: 2.8241x; 2.8241x over previous
import functools

import jax
import jax.numpy as jnp
from jax import lax
from jax.experimental import pallas as pl
from jax.experimental.pallas import tpu as pltpu
from jax.experimental.pallas import tpu_sc as plsc

D = 64
K = 10
L = 16
NC = 2
NS = 16
NW = NC * NS
CB = 128


def _sc_scores(target_idx, context_idx, negative_idx_t, target_embeddings,
               context_embeddings):
    B = target_idx.shape[0]
    bpw = B // NW
    nchunks = bpw // CB

    mesh = plsc.VectorSubcoreMesh(
        core_axis_name="c", subcore_axis_name="s", num_cores=NC,
        num_subcores=NS)

    @functools.partial(
        pl.kernel,
        out_type=(
            jax.ShapeDtypeStruct((B,), jnp.float32),
            jax.ShapeDtypeStruct((NW, K * bpw), jnp.float32),
        ),
        mesh=mesh,
        scratch_types=[
            pltpu.VMEM((CB,), jnp.int32),
            pltpu.VMEM((CB,), jnp.int32),
            pltpu.VMEM((K, CB), jnp.int32),
            pltpu.VMEM((CB, D), jnp.float32),
            pltpu.VMEM((CB, D), jnp.float32),
            pltpu.VMEM((K, CB, D), jnp.float32),
            pltpu.VMEM((bpw + L,), jnp.float32),
            pltpu.VMEM((K * bpw + L,), jnp.float32),
            pltpu.SemaphoreType.DMA,
        ],
        compiler_params=pltpu.CompilerParams(
            needs_layout_passes=False, use_tc_tiling_on_sc=False),
    )
    def sc_kernel(tidx_hbm, cidx_hbm, nidx_hbm, temb_hbm, cemb_hbm,
                  pos_hbm, neg_hbm,
                  tiv, civ, niv, trows, crows, nrows, posv, negv, sem):
        wid = lax.axis_index("s") * NC + lax.axis_index("c")
        base = wid * bpw
        last_lane = lax.iota(jnp.int32, L) == (L - 1)

        for c in range(nchunks):
            cb0 = c * CB
            pltpu.sync_copy(tidx_hbm.at[pl.ds(base + cb0, CB)], tiv)
            pltpu.sync_copy(cidx_hbm.at[pl.ds(base + cb0, CB)], civ)
            pltpu.sync_copy(nidx_hbm.at[:, pl.ds(base + cb0, CB)], niv)

            copies = [
                pltpu.async_copy(temb_hbm.at[tiv], trows, sem),
                pltpu.async_copy(cemb_hbm.at[civ], crows, sem),
            ]
            for k in range(K):
                copies.append(
                    pltpu.async_copy(cemb_hbm.at[niv.at[k]], nrows.at[k], sem))
            for cp in copies:
                cp.wait()

            def body(i, carry):
                t = [trows[i, pl.ds(j * L, L)] for j in range(D // L)]
                cv = [crows[i, pl.ds(j * L, L)] for j in range(D // L)]
                p = t[0] * cv[0] + t[1] * cv[1] + t[2] * cv[2] + t[3] * cv[3]
                plsc.store_compressed(posv.at[pl.ds(cb0 + i, L)],
                                      plsc.cumsum(p), mask=last_lane)
                for k in range(K):
                    n = [nrows[k, i, pl.ds(j * L, L)] for j in range(D // L)]
                    q = n[0] * t[0] + n[1] * t[1] + n[2] * t[2] + n[3] * t[3]
                    plsc.store_compressed(negv.at[pl.ds(k * bpw + cb0 + i, L)],
                                          plsc.cumsum(q), mask=last_lane)
                return carry

            lax.fori_loop(0, CB, body, 0)

        pltpu.sync_copy(posv.at[pl.ds(0, bpw)], pos_hbm.at[pl.ds(base, bpw)])
        pltpu.sync_copy(negv.at[pl.ds(0, K * bpw)], neg_hbm.at[wid])

    return sc_kernel(target_idx, context_idx, negative_idx_t,
                     target_embeddings, context_embeddings)


def _loss_tc(pos_scores, neg_scores):
    pos2 = pos_scores.reshape(-1, 128)
    neg2 = neg_scores.reshape(-1, 128)

    def body(pos_ref, neg_ref, out_ref):
        p = pos_ref[...]
        n = neg_ref[...]
        ls_p = jnp.minimum(p, 0.0) - jnp.log1p(jnp.exp(-jnp.abs(p)))
        ls_n = jnp.minimum(-n, 0.0) - jnp.log1p(jnp.exp(-jnp.abs(n)))
        out_ref[0, 0] = -(jnp.mean(ls_p) + jnp.mean(ls_n))

    out = pl.pallas_call(
        body,
        out_shape=jax.ShapeDtypeStruct((1, 1), jnp.float32),
        out_specs=pl.BlockSpec(memory_space=pltpu.SMEM),
    )(pos2, neg2)
    return out[0, 0]


def kernel(target_idx, context_idx, negative_idx, target_embeddings,
           context_embeddings):
    negative_idx_t = negative_idx.T
    pos_scores, neg_scores = _sc_scores(
        target_idx, context_idx, negative_idx_t, target_embeddings,
        context_embeddings)
    return _loss_tc(pos_scores, neg_scores)
